# Initial kernel scaffold; baseline (speedup 1.0000x reference)
import jax
import jax.numpy as jnp
from jax.experimental import pallas as pl


def kernel(x, edge_indexs, edge_attrs, W_lin, b_lin, W_convs, b_convs):
    raise NotImplementedError("write your pallas kernel here")


# speedup vs baseline: 16.0465x; 16.0465x over previous
import dataclasses
import functools

import jax
import jax.numpy as jnp
from jax import lax
from jax.experimental import pallas as pl
from jax.experimental.pallas import tpu as pltpu
from jax.experimental.pallas import tpu_sc as plsc

N = 10000
E = 320000
ALPHA = 2
D = 128

NC = 2
NS = 16
NW = NC * NS
L = 16
EPT = E // NW
C = 80
NCHUNK = EPT // C
STRIPE = 632
LAST_STRIPE = N - (NS - 1) * STRIPE

_mesh = plsc.VectorSubcoreMesh(core_axis_name="c", subcore_axis_name="s")

_sc_params = pltpu.CompilerParams()
if "needs_layout_passes" in pltpu.CompilerParams.__dataclass_fields__:
  _sc_params = dataclasses.replace(_sc_params, needs_layout_passes=False)


@functools.partial(
    pl.kernel,
    out_type=jax.ShapeDtypeStruct((NW * ALPHA * N,), jnp.float32),
    mesh=_mesh,
    scratch_types=[
        pltpu.VMEM((ALPHA * N,), jnp.float32),
        pltpu.VMEM((2000,), jnp.int32),
        pltpu.VMEM((2000,), jnp.float32),
        pltpu.SemaphoreType.DMA,
    ],
    compiler_params=_sc_params,
)
def _deg_kernel(cols_hbm, ws_hbm, degp_hbm, degb, cbuf, wbuf, sem):
  cid = lax.axis_index("c")
  sid = lax.axis_index("s")
  wid = cid * NS + sid

  zero = jnp.zeros((L,), jnp.float32)

  @pl.loop(0, ALPHA * N // L)
  def _(g):
    degb[pl.ds(g * L, L)] = zero

  for a in range(ALPHA):
    for k in range(EPT // 2000):
      off = a * E + wid * EPT + k * 2000
      pltpu.async_copy(cols_hbm.at[pl.ds(off, 2000)], cbuf, sem).wait()
      pltpu.async_copy(ws_hbm.at[pl.ds(off, 2000)], wbuf, sem).wait()

      @pl.loop(0, 2000 // L)
      def _(g):
        c16 = cbuf[pl.ds(g * L, L)] + a * N
        w16 = wbuf[pl.ds(g * L, L)]
        plsc.addupdate_scatter(degb, [c16], w16)

  pltpu.sync_copy(degb, degp_hbm.at[pl.ds(wid * ALPHA * N, ALPHA * N)])


def _dinv_body(degp_ref, dinv_ref):
  d = jnp.sum(degp_ref[...], axis=0)
  dinv_ref[...] = jnp.where(d > 0.0, lax.rsqrt(jnp.where(d > 0.0, d, 1.0)), 0.0)


_dinv_tc = pl.pallas_call(
    _dinv_body,
    out_shape=jax.ShapeDtypeStruct((ALPHA * N,), jnp.float32),
)


def _make_msg_kernel(aoff):
  @functools.partial(
      pl.kernel,
      out_type=jax.ShapeDtypeStruct((NC * N, D), jnp.float32),
      mesh=_mesh,
      scratch_types=[
          pltpu.VMEM((EPT,), jnp.int32),
          pltpu.VMEM((EPT,), jnp.int32),
          pltpu.VMEM((EPT,), jnp.float32),
          pltpu.VMEM((N,), jnp.float32),
          pltpu.VMEM((C,), jnp.int32),
          pltpu.VMEM((C,), jnp.int32),
          pltpu.VMEM((C, D), jnp.float32),
          pltpu.VMEM_SHARED((N, D), jnp.float32),
          pltpu.SemaphoreType.DMA,
      ],
      compiler_params=_sc_params,
  )
  def _msg_kernel(mt_hbm, row_hbm, col_hbm, w_hbm, dinv_hbm, out_hbm,
                  row_s, col_s, w_s, dinv_v, row_c, col_c, rows_v,
                  acc_sh, sem):
    cid = lax.axis_index("c")
    sid = lax.axis_index("s")
    wid = cid * NS + sid
    nrows = jnp.where(sid == NS - 1, LAST_STRIPE, STRIPE)

    zero = jnp.zeros((L,), jnp.float32)

    @pl.loop(0, C * D // L)
    def _(g):
      rows_v[g * L // D, pl.ds((g * L) % D, L)] = zero

    @pl.loop(0, nrows // 8)
    def _(k):
      pltpu.sync_copy(rows_v.at[pl.ds(0, 8)],
                      acc_sh.at[pl.ds(sid * STRIPE + k * 8, 8)])

    pltpu.async_copy(row_hbm.at[pl.ds(wid * EPT, EPT)], row_s, sem).wait()
    pltpu.async_copy(col_hbm.at[pl.ds(wid * EPT, EPT)], col_s, sem).wait()
    pltpu.async_copy(w_hbm.at[pl.ds(wid * EPT, EPT)], w_s, sem).wait()
    pltpu.async_copy(dinv_hbm.at[pl.ds(aoff, N)], dinv_v, sem).wait()

    @pl.loop(0, EPT // L)
    def _(g):
      r16 = row_s[pl.ds(g * L, L)]
      c16 = col_s[pl.ds(g * L, L)]
      dr = plsc.load_gather(dinv_v, [r16])
      dc = plsc.load_gather(dinv_v, [c16])
      w_s[pl.ds(g * L, L)] = dr * w_s[pl.ds(g * L, L)] * dc

    plsc.subcore_barrier()

    @pl.loop(0, NCHUNK)
    def _(it):
      for g in range(C // L):
        row_c[pl.ds(g * L, L)] = row_s[pl.ds(it * C + g * L, L)]
        col_c[pl.ds(g * L, L)] = col_s[pl.ds(it * C + g * L, L)]

      pltpu.async_copy(mt_hbm.at[row_c], rows_v, sem).wait()

      @pl.loop(0, C)
      def _(e):
        s = plsc.load_gather(w_s, [jnp.full((L,), it * C + e, jnp.int32)])
        for j in range(D // L):
          rows_v[e, pl.ds(j * L, L)] = rows_v[e, pl.ds(j * L, L)] * s

      pltpu.sync_copy(rows_v, acc_sh.at[col_c], add=True)

    plsc.subcore_barrier()

    pltpu.sync_copy(acc_sh.at[pl.ds(sid * STRIPE, nrows)],
                    out_hbm.at[pl.ds(cid * N + sid * STRIPE, nrows)])

  return _msg_kernel


_msg_kernels = [_make_msg_kernel(a * N) for a in range(ALPHA)]


RB = 1000


def _stage1_body(x_ref, wl_ref, bl_ref, w1_ref, h0_ref, m1_ref):
  h = jnp.dot(x_ref[...], wl_ref[...], preferred_element_type=jnp.float32)
  h = jnp.maximum(h + bl_ref[...], 0.0)
  h0_ref[...] = h
  m1_ref[...] = jnp.dot(h, w1_ref[...], preferred_element_type=jnp.float32)


_stage1 = pl.pallas_call(
    _stage1_body,
    grid=(N // RB,),
    in_specs=[
        pl.BlockSpec((RB, D), lambda i: (i, 0)),
        pl.BlockSpec((D, D), lambda i: (0, 0)),
        pl.BlockSpec((1, D), lambda i: (0, 0)),
        pl.BlockSpec((D, D), lambda i: (0, 0)),
    ],
    out_specs=[
        pl.BlockSpec((RB, D), lambda i: (i, 0)),
        pl.BlockSpec((RB, D), lambda i: (i, 0)),
    ],
    out_shape=[
        jax.ShapeDtypeStruct((N, D), jnp.float32),
        jax.ShapeDtypeStruct((N, D), jnp.float32),
    ],
)


def _stage2_body(acc_ref, b_ref, w2_ref, h_ref, m2_ref):
  s = acc_ref[0] + acc_ref[1] + b_ref[...]
  h = jnp.maximum(s, 0.0)
  h_ref[...] = h
  m2_ref[...] = jnp.dot(h, w2_ref[...], preferred_element_type=jnp.float32)


_stage2 = pl.pallas_call(
    _stage2_body,
    grid=(N // RB,),
    in_specs=[
        pl.BlockSpec((NC, RB, D), lambda i: (0, i, 0)),
        pl.BlockSpec((1, D), lambda i: (0, 0)),
        pl.BlockSpec((D, D), lambda i: (0, 0)),
    ],
    out_specs=[
        pl.BlockSpec((RB, D), lambda i: (i, 0)),
        pl.BlockSpec((RB, D), lambda i: (i, 0)),
    ],
    out_shape=[
        jax.ShapeDtypeStruct((N, D), jnp.float32),
        jax.ShapeDtypeStruct((N, D), jnp.float32),
    ],
)


def _stage3_body(acc_ref, b_ref, h_ref):
  h_ref[...] = jnp.maximum(acc_ref[0] + acc_ref[1] + b_ref[...], 0.0)


_stage3 = pl.pallas_call(
    _stage3_body,
    grid=(N // RB,),
    in_specs=[
        pl.BlockSpec((NC, RB, D), lambda i: (0, i, 0)),
        pl.BlockSpec((1, D), lambda i: (0, 0)),
    ],
    out_specs=pl.BlockSpec((RB, D), lambda i: (i, 0)),
    out_shape=jax.ShapeDtypeStruct((N, D), jnp.float32),
)


def kernel(x, edge_indexs, edge_attrs, W_lin, b_lin, W_convs, b_convs):
  edge_indexs = edge_indexs.astype(jnp.int32)
  cols_flat = edge_indexs[:, 1, :].reshape(-1)
  ws_flat = edge_attrs.astype(jnp.float32).reshape(-1)

  degp = _deg_kernel(cols_flat, ws_flat)
  dinv = _dinv_tc(degp.reshape(NW, ALPHA * N))

  h0, m1 = _stage1(x, W_lin, b_lin.reshape(1, D), W_convs[0])

  accp1 = _msg_kernels[0](m1, edge_indexs[0, 0], edge_indexs[0, 1],
                          edge_attrs[0], dinv)
  accp1 = accp1.reshape(NC, N, D)

  h1, m2 = _stage2(accp1, b_convs[0].reshape(1, D), W_convs[1])

  accp2 = _msg_kernels[1](m2, edge_indexs[1, 0], edge_indexs[1, 1],
                          edge_attrs[1], dinv)
  accp2 = accp2.reshape(NC, N, D)

  h2 = _stage3(accp2, b_convs[1].reshape(1, D))

  return (h0, h1, h2)


# speedup vs baseline: 24.0018x; 1.4958x over previous
import dataclasses
import functools

import jax
import jax.numpy as jnp
from jax import lax
from jax.experimental import pallas as pl
from jax.experimental.pallas import tpu as pltpu
from jax.experimental.pallas import tpu_sc as plsc

N = 10000
E = 320000
ALPHA = 2
D = 128

NC = 2
NS = 16
NW = NC * NS
L = 16
EPT = E // NW
C = 80
NCHUNK = EPT // C
STRIPE = 632
LAST_STRIPE = N - (NS - 1) * STRIPE

_mesh = plsc.VectorSubcoreMesh(core_axis_name="c", subcore_axis_name="s")

_sc_params = pltpu.CompilerParams()
if "needs_layout_passes" in pltpu.CompilerParams.__dataclass_fields__:
  _sc_params = dataclasses.replace(_sc_params, needs_layout_passes=False)


@functools.partial(
    pl.kernel,
    out_type=jax.ShapeDtypeStruct((NW * ALPHA * N,), jnp.float32),
    mesh=_mesh,
    scratch_types=[
        pltpu.VMEM((ALPHA * N,), jnp.float32),
        pltpu.VMEM((EPT,), jnp.int32),
        pltpu.VMEM((EPT,), jnp.float32),
        pltpu.VMEM((EPT,), jnp.int32),
        pltpu.VMEM((EPT,), jnp.float32),
        pltpu.SemaphoreType.DMA,
        pltpu.SemaphoreType.DMA,
    ],
    compiler_params=_sc_params,
)
def _deg_kernel(cols_hbm, ws_hbm, degp_hbm, degb, c0, w0, c1, w1, s0, s1):
  cid = lax.axis_index("c")
  sid = lax.axis_index("s")
  wid = cid * NS + sid

  cbufs, wbufs, sems = (c0, c1), (w0, w1), (s0, s1)
  for a in range(ALPHA):
    off = a * E + wid * EPT
    pltpu.async_copy(cols_hbm.at[pl.ds(off, EPT)], cbufs[a], sems[a])
    pltpu.async_copy(ws_hbm.at[pl.ds(off, EPT)], wbufs[a], sems[a])

  zero = jnp.zeros((L,), jnp.float32)

  @pl.loop(0, ALPHA * N // L)
  def _(g):
    degb[pl.ds(g * L, L)] = zero

  for a in range(ALPHA):
    pltpu.make_async_copy(cols_hbm.at[pl.ds(0, EPT)], cbufs[a], sems[a]).wait()
    pltpu.make_async_copy(ws_hbm.at[pl.ds(0, EPT)], wbufs[a], sems[a]).wait()

    @pl.loop(0, EPT // L)
    def _(g):
      c16 = cbufs[a][pl.ds(g * L, L)] + a * N
      w16 = wbufs[a][pl.ds(g * L, L)]
      plsc.addupdate_scatter(degb, [c16], w16)

  pltpu.sync_copy(degb, degp_hbm.at[pl.ds(wid * ALPHA * N, ALPHA * N)])


def _dinv_body(degp_ref, dinv_ref):
  d = jnp.sum(degp_ref[...], axis=0)
  dinv_ref[...] = jnp.where(d > 0.0, lax.rsqrt(jnp.where(d > 0.0, d, 1.0)), 0.0)


_dinv_tc = pl.pallas_call(
    _dinv_body,
    out_shape=jax.ShapeDtypeStruct((ALPHA * N,), jnp.float32),
)


PK = 3 * C


def _make_msg_kernel(aoff):
  @functools.partial(
      pl.kernel,
      out_type=jax.ShapeDtypeStruct((NC * N, D), jnp.float32),
      mesh=_mesh,
      scratch_types=[
          pltpu.VMEM((N,), jnp.float32),
          pltpu.VMEM((PK,), jnp.int32),
          pltpu.VMEM((PK,), jnp.int32),
          pltpu.VMEM((PK,), jnp.int32),
          pltpu.VMEM((C, D), jnp.float32),
          pltpu.VMEM((C, D), jnp.float32),
          pltpu.VMEM((C,), jnp.int32),
          pltpu.VMEM((C,), jnp.int32),
          pltpu.VMEM((C,), jnp.int32),
          pltpu.VMEM((C,), jnp.int32),
          pltpu.VMEM((C,), jnp.float32),
          pltpu.VMEM_SHARED((N, D), jnp.float32),
          pltpu.SemaphoreType.DMA,
          pltpu.SemaphoreType.DMA,
          pltpu.SemaphoreType.DMA,
          pltpu.SemaphoreType.DMA,
          pltpu.SemaphoreType.DMA,
          pltpu.SemaphoreType.DMA,
          pltpu.SemaphoreType.DMA,
          pltpu.SemaphoreType.DMA,
      ],
      compiler_params=_sc_params,
  )
  def _msg_kernel(mt_hbm, edges_hbm, dinv_hbm, out_hbm,
                  dinv_v, e0, e1, e2, r0, r1, c0, c1, rw0, rw1, norm_c,
                  acc_sh, se0, se1, se2, sg0, sg1, ss0, ss1, sd):
    cid = lax.axis_index("c")
    sid = lax.axis_index("s")
    wid = cid * NS + sid
    nrows = jnp.where(sid == NS - 1, LAST_STRIPE, STRIPE)
    cbase = wid * NCHUNK

    ebufs, ses = (e0, e1, e2), (se0, se1, se2)
    rbufs, sgs = (r0, r1), (sg0, sg1)
    cbufs, sss = (c0, c1), (ss0, ss1)
    rwbufs = (rw0, rw1)

    pltpu.async_copy(dinv_hbm.at[pl.ds(aoff, N)], dinv_v, sd)

    def issue_e(i, k):
      pltpu.async_copy(edges_hbm.at[pl.ds((cbase + i) * PK, PK)],
                       ebufs[k % 3], ses[k % 3])

    def wait_e(k):
      pltpu.make_async_copy(edges_hbm.at[pl.ds(0, PK)],
                            ebufs[k % 3], ses[k % 3]).wait()

    def stage_rows(k):
      eb, rwb = ebufs[k % 3], rwbufs[k % 2]
      for g in range(C // L):
        rwb[pl.ds(g * L, L)] = eb[pl.ds(g * L, L)]

    def issue_g(k):
      pltpu.async_copy(mt_hbm.at[rwbufs[k % 2]], rbufs[k % 2], sgs[k % 2])

    def wait_g(k):
      pltpu.make_async_copy(mt_hbm.at[rwbufs[k % 2]],
                            rbufs[k % 2], sgs[k % 2]).wait()

    def issue_s(k):
      pltpu.async_copy(rbufs[k % 2], acc_sh.at[cbufs[k % 2]],
                       sss[k % 2], add=True)

    def wait_s(k):
      pltpu.make_async_copy(rbufs[k % 2], acc_sh.at[cbufs[k % 2]],
                            sss[k % 2]).wait()

    def proc_norm(k):
      eb, cb = ebufs[k % 3], cbufs[k % 2]
      for g in range(C // L):
        r16 = eb[pl.ds(g * L, L)]
        c16 = eb[pl.ds(C + g * L, L)]
        w16 = plsc.bitcast(eb[pl.ds(2 * C + g * L, L)], jnp.float32)
        cb[pl.ds(g * L, L)] = c16
        norm_c[pl.ds(g * L, L)] = (plsc.load_gather(dinv_v, [r16]) * w16
                                   * plsc.load_gather(dinv_v, [c16]))

    def proc_scale(k):
      rb = rbufs[k % 2]

      @pl.loop(0, C)
      def _(e):
        s = plsc.load_gather(norm_c, [jnp.full((L,), e, jnp.int32)])
        for j in range(D // L):
          rb[e, pl.ds(j * L, L)] = rb[e, pl.ds(j * L, L)] * s

    def body(i, k, first=False, next_e=True, next_g=True):
      wait_g(k)
      proc_norm(k)
      if next_e:
        issue_e(i + 3, k)
      if not first:
        wait_s(k + 1)
      if next_g:
        wait_e(k + 1)
        stage_rows(k + 1)
        issue_g(k + 1)
      proc_scale(k)
      issue_s(k)

    zero = jnp.zeros((L,), jnp.float32)

    @pl.loop(0, C * D // L)
    def _(g):
      r0[g * L // D, pl.ds((g * L) % D, L)] = zero

    @pl.loop(0, nrows // 8)
    def _(k):
      pltpu.sync_copy(r0.at[pl.ds(0, 8)],
                      acc_sh.at[pl.ds(sid * STRIPE + k * 8, 8)])

    pltpu.make_async_copy(dinv_hbm.at[pl.ds(0, N)], dinv_v, sd).wait()
    plsc.subcore_barrier()

    issue_e(0, 0)
    issue_e(1, 1)
    issue_e(2, 2)
    wait_e(0)
    stage_rows(0)
    issue_g(0)
    body(0, 0, first=True)
    for i in range(1, 5):
      body(i, i)

    @pl.loop(0, 19)
    def _(t):
      i0 = 5 + t * 6
      for k in range(6):
        body(i0 + k, 5 + k)

    for i in range(119, NCHUNK):
      body(i, i, next_e=(i + 3 < NCHUNK), next_g=(i + 1 < NCHUNK))
    wait_s(NCHUNK - 1)

    plsc.subcore_barrier()

    pltpu.sync_copy(acc_sh.at[pl.ds(sid * STRIPE, nrows)],
                    out_hbm.at[pl.ds(cid * N + sid * STRIPE, nrows)])

  return _msg_kernel


_msg_kernels = [_make_msg_kernel(a * N) for a in range(ALPHA)]


RB = 1000


def _stage1_body(x_ref, wl_ref, bl_ref, w1_ref, h0_ref, m1_ref):
  h = jnp.dot(x_ref[...], wl_ref[...], preferred_element_type=jnp.float32)
  h = jnp.maximum(h + bl_ref[...], 0.0)
  h0_ref[...] = h
  m1_ref[...] = jnp.dot(h, w1_ref[...], preferred_element_type=jnp.float32)


_stage1 = pl.pallas_call(
    _stage1_body,
    grid=(N // RB,),
    in_specs=[
        pl.BlockSpec((RB, D), lambda i: (i, 0)),
        pl.BlockSpec((D, D), lambda i: (0, 0)),
        pl.BlockSpec((1, D), lambda i: (0, 0)),
        pl.BlockSpec((D, D), lambda i: (0, 0)),
    ],
    out_specs=[
        pl.BlockSpec((RB, D), lambda i: (i, 0)),
        pl.BlockSpec((RB, D), lambda i: (i, 0)),
    ],
    out_shape=[
        jax.ShapeDtypeStruct((N, D), jnp.float32),
        jax.ShapeDtypeStruct((N, D), jnp.float32),
    ],
)


def _stage2_body(acc_ref, b_ref, w2_ref, h_ref, m2_ref):
  s = acc_ref[0] + acc_ref[1] + b_ref[...]
  h = jnp.maximum(s, 0.0)
  h_ref[...] = h
  m2_ref[...] = jnp.dot(h, w2_ref[...], preferred_element_type=jnp.float32)


_stage2 = pl.pallas_call(
    _stage2_body,
    grid=(N // RB,),
    in_specs=[
        pl.BlockSpec((NC, RB, D), lambda i: (0, i, 0)),
        pl.BlockSpec((1, D), lambda i: (0, 0)),
        pl.BlockSpec((D, D), lambda i: (0, 0)),
    ],
    out_specs=[
        pl.BlockSpec((RB, D), lambda i: (i, 0)),
        pl.BlockSpec((RB, D), lambda i: (i, 0)),
    ],
    out_shape=[
        jax.ShapeDtypeStruct((N, D), jnp.float32),
        jax.ShapeDtypeStruct((N, D), jnp.float32),
    ],
)


def _stage3_body(acc_ref, b_ref, h_ref):
  h_ref[...] = jnp.maximum(acc_ref[0] + acc_ref[1] + b_ref[...], 0.0)


_stage3 = pl.pallas_call(
    _stage3_body,
    grid=(N // RB,),
    in_specs=[
        pl.BlockSpec((NC, RB, D), lambda i: (0, i, 0)),
        pl.BlockSpec((1, D), lambda i: (0, 0)),
    ],
    out_specs=pl.BlockSpec((RB, D), lambda i: (i, 0)),
    out_shape=jax.ShapeDtypeStruct((N, D), jnp.float32),
)


def kernel(x, edge_indexs, edge_attrs, W_lin, b_lin, W_convs, b_convs):
  edge_indexs = edge_indexs.astype(jnp.int32)
  edge_attrs = edge_attrs.astype(jnp.float32)
  cols_flat = edge_indexs[:, 1, :].reshape(-1)
  ws_flat = edge_attrs.reshape(-1)

  wbits = jax.lax.bitcast_convert_type(edge_attrs, jnp.int32)
  packed = [
      jnp.stack([edge_indexs[a, 0].reshape(-1, C),
                 edge_indexs[a, 1].reshape(-1, C),
                 wbits[a].reshape(-1, C)], axis=1).reshape(-1)
      for a in range(ALPHA)
  ]

  degp = _deg_kernel(cols_flat, ws_flat)
  dinv = _dinv_tc(degp.reshape(NW, ALPHA * N))

  h0, m1 = _stage1(x, W_lin, b_lin.reshape(1, D), W_convs[0])

  accp1 = _msg_kernels[0](m1, packed[0], dinv).reshape(NC, N, D)

  h1, m2 = _stage2(accp1, b_convs[0].reshape(1, D), W_convs[1])

  accp2 = _msg_kernels[1](m2, packed[1], dinv).reshape(NC, N, D)

  h2 = _stage3(accp2, b_convs[1].reshape(1, D))

  return (h0, h1, h2)


# speedup vs baseline: 27.3168x; 1.1381x over previous
import dataclasses
import functools

import jax
import jax.numpy as jnp
from jax import lax
from jax.experimental import pallas as pl
from jax.experimental.pallas import tpu as pltpu
from jax.experimental.pallas import tpu_sc as plsc

N = 10000
E = 320000
ALPHA = 2
D = 128

NC = 2
NS = 16
NW = NC * NS
L = 16
EPT = E // NW
C = 80
NCHUNK = EPT // C
STRIPE = 632
LAST_STRIPE = N - (NS - 1) * STRIPE

_mesh = plsc.VectorSubcoreMesh(core_axis_name="c", subcore_axis_name="s")

_sc_params = pltpu.CompilerParams()
if "needs_layout_passes" in pltpu.CompilerParams.__dataclass_fields__:
  _sc_params = dataclasses.replace(_sc_params, needs_layout_passes=False)


@functools.partial(
    pl.kernel,
    out_type=jax.ShapeDtypeStruct((NW * ALPHA * N,), jnp.float32),
    mesh=_mesh,
    scratch_types=[
        pltpu.VMEM((ALPHA * N,), jnp.float32),
        pltpu.VMEM((EPT,), jnp.int32),
        pltpu.VMEM((EPT,), jnp.float32),
        pltpu.VMEM((EPT,), jnp.int32),
        pltpu.VMEM((EPT,), jnp.float32),
        pltpu.SemaphoreType.DMA,
        pltpu.SemaphoreType.DMA,
    ],
    compiler_params=_sc_params,
)
def _deg_kernel(cols_hbm, ws_hbm, degp_hbm, degb, c0, w0, c1, w1, s0, s1):
  cid = lax.axis_index("c")
  sid = lax.axis_index("s")
  wid = cid * NS + sid

  cbufs, wbufs, sems = (c0, c1), (w0, w1), (s0, s1)
  for a in range(ALPHA):
    off = a * E + wid * EPT
    pltpu.async_copy(cols_hbm.at[pl.ds(off, EPT)], cbufs[a], sems[a])
    pltpu.async_copy(ws_hbm.at[pl.ds(off, EPT)], wbufs[a], sems[a])

  zero = jnp.zeros((L,), jnp.float32)

  @pl.loop(0, ALPHA * N // L)
  def _(g):
    degb[pl.ds(g * L, L)] = zero

  for a in range(ALPHA):
    pltpu.make_async_copy(cols_hbm.at[pl.ds(0, EPT)], cbufs[a], sems[a]).wait()
    pltpu.make_async_copy(ws_hbm.at[pl.ds(0, EPT)], wbufs[a], sems[a]).wait()

    @pl.loop(0, EPT // L)
    def _(g):
      c16 = cbufs[a][pl.ds(g * L, L)] + a * N
      w16 = wbufs[a][pl.ds(g * L, L)]
      plsc.addupdate_scatter(degb, [c16], w16)

  pltpu.sync_copy(degb, degp_hbm.at[pl.ds(wid * ALPHA * N, ALPHA * N)])


def _dinv_body(degp_ref, dinv_ref):
  d = jnp.sum(degp_ref[...], axis=0)
  dinv_ref[...] = jnp.where(d > 0.0, lax.rsqrt(jnp.where(d > 0.0, d, 1.0)), 0.0)


_dinv_tc = pl.pallas_call(
    _dinv_body,
    out_shape=jax.ShapeDtypeStruct((ALPHA * N,), jnp.float32),
)


PK = 3 * C


def _make_msg_kernel(aoff):
  @functools.partial(
      pl.kernel,
      out_type=jax.ShapeDtypeStruct((NC * N, D), jnp.float32),
      mesh=_mesh,
      scratch_types=[
          pltpu.VMEM((N,), jnp.float32),
          pltpu.VMEM((PK,), jnp.int32),
          pltpu.VMEM((PK,), jnp.int32),
          pltpu.VMEM((PK,), jnp.int32),
          pltpu.VMEM((C, D), jnp.float32),
          pltpu.VMEM((C, D), jnp.float32),
          pltpu.VMEM((C, D), jnp.float32),
          pltpu.VMEM((C,), jnp.int32),
          pltpu.VMEM((C,), jnp.int32),
          pltpu.VMEM((C,), jnp.int32),
          pltpu.VMEM((C,), jnp.int32),
          pltpu.VMEM((C,), jnp.int32),
          pltpu.VMEM((C,), jnp.int32),
          pltpu.VMEM((C,), jnp.float32),
          pltpu.VMEM_SHARED((N, D), jnp.float32),
          pltpu.SemaphoreType.DMA,
          pltpu.SemaphoreType.DMA,
          pltpu.SemaphoreType.DMA,
          pltpu.SemaphoreType.DMA,
          pltpu.SemaphoreType.DMA,
          pltpu.SemaphoreType.DMA,
          pltpu.SemaphoreType.DMA,
          pltpu.SemaphoreType.DMA,
          pltpu.SemaphoreType.DMA,
          pltpu.SemaphoreType.DMA,
      ],
      compiler_params=_sc_params,
  )
  def _msg_kernel(mt_hbm, edges_hbm, dinv_hbm, out_hbm,
                  dinv_v, e0, e1, e2, r0, r1, r2, c0, c1, c2,
                  rw0, rw1, rw2, norm_c, acc_sh,
                  se0, se1, se2, sg0, sg1, sg2, ss0, ss1, ss2, sd):
    cid = lax.axis_index("c")
    sid = lax.axis_index("s")
    wid = cid * NS + sid
    nrows = jnp.where(sid == NS - 1, LAST_STRIPE, STRIPE)
    cbase = wid * NCHUNK

    ebufs, ses = (e0, e1, e2), (se0, se1, se2)
    rbufs, sgs = (r0, r1, r2), (sg0, sg1, sg2)
    cbufs, sss = (c0, c1, c2), (ss0, ss1, ss2)
    rwbufs = (rw0, rw1, rw2)

    pltpu.async_copy(dinv_hbm.at[pl.ds(aoff, N)], dinv_v, sd)

    def issue_e(i, k):
      pltpu.async_copy(edges_hbm.at[pl.ds((cbase + i) * PK, PK)],
                       ebufs[k % 3], ses[k % 3])

    def wait_e(k):
      pltpu.make_async_copy(edges_hbm.at[pl.ds(0, PK)],
                            ebufs[k % 3], ses[k % 3]).wait()

    def stage_rows(k):
      eb, rwb = ebufs[k % 3], rwbufs[k % 3]
      for g in range(C // L):
        rwb[pl.ds(g * L, L)] = eb[pl.ds(g * L, L)]

    def issue_g(k):
      pltpu.async_copy(mt_hbm.at[rwbufs[k % 3]], rbufs[k % 3], sgs[k % 3])

    def wait_g(k):
      pltpu.make_async_copy(mt_hbm.at[rwbufs[k % 3]],
                            rbufs[k % 3], sgs[k % 3]).wait()

    def issue_s(k):
      pltpu.async_copy(rbufs[k % 3], acc_sh.at[cbufs[k % 3]],
                       sss[k % 3], add=True)

    def wait_s(k):
      pltpu.make_async_copy(rbufs[k % 3], acc_sh.at[cbufs[k % 3]],
                            sss[k % 3]).wait()

    def proc_norm(k):
      eb, cb = ebufs[k % 3], cbufs[k % 3]
      for g in range(C // L):
        r16 = eb[pl.ds(g * L, L)]
        c16 = eb[pl.ds(C + g * L, L)]
        w16 = plsc.bitcast(eb[pl.ds(2 * C + g * L, L)], jnp.float32)
        cb[pl.ds(g * L, L)] = c16
        norm_c[pl.ds(g * L, L)] = (plsc.load_gather(dinv_v, [r16]) * w16
                                   * plsc.load_gather(dinv_v, [c16]))

    def proc_scale(k):
      rb = rbufs[k % 3]

      @pl.loop(0, C)
      def _(e):
        s = plsc.load_gather(norm_c, [jnp.full((L,), e, jnp.int32)])
        for j in range(D // L):
          rb[e, pl.ds(j * L, L)] = rb[e, pl.ds(j * L, L)] * s

    def body(i, k, drain_s=True, next_e=True, next_g=True):
      wait_g(k)
      proc_norm(k)
      if next_e:
        issue_e(i + 3, k)
      proc_scale(k)
      issue_s(k)
      if drain_s:
        wait_s(k + 2)
      if next_g:
        wait_e(k + 2)
        stage_rows(k + 2)
        issue_g(k + 2)

    zero = jnp.zeros((L,), jnp.float32)

    @pl.loop(0, C * D // L)
    def _(g):
      r0[g * L // D, pl.ds((g * L) % D, L)] = zero

    @pl.loop(0, nrows // 8)
    def _(k):
      pltpu.sync_copy(r0.at[pl.ds(0, 8)],
                      acc_sh.at[pl.ds(sid * STRIPE + k * 8, 8)])

    pltpu.make_async_copy(dinv_hbm.at[pl.ds(0, N)], dinv_v, sd).wait()
    plsc.subcore_barrier()

    issue_e(0, 0)
    issue_e(1, 1)
    issue_e(2, 2)
    wait_e(0)
    stage_rows(0)
    issue_g(0)
    wait_e(1)
    stage_rows(1)
    issue_g(1)
    body(0, 0, drain_s=False)
    body(1, 1)

    @pl.loop(0, 40)
    def _(t):
      i0 = 2 + t * 3
      for k in range(3):
        body(i0 + k, 2 + k)

    for i in range(122, NCHUNK):
      body(i, i, next_e=(i + 3 < NCHUNK), next_g=(i + 2 < NCHUNK))
    wait_s(NCHUNK - 1)

    plsc.subcore_barrier()

    pltpu.sync_copy(acc_sh.at[pl.ds(sid * STRIPE, nrows)],
                    out_hbm.at[pl.ds(cid * N + sid * STRIPE, nrows)])

  return _msg_kernel


_msg_kernels = [_make_msg_kernel(a * N) for a in range(ALPHA)]


RB = 1000


def _stage1_body(x_ref, wl_ref, bl_ref, w1_ref, h0_ref, m1_ref):
  h = jnp.dot(x_ref[...], wl_ref[...], preferred_element_type=jnp.float32)
  h = jnp.maximum(h + bl_ref[...], 0.0)
  h0_ref[...] = h
  m1_ref[...] = jnp.dot(h, w1_ref[...], preferred_element_type=jnp.float32)


_stage1 = pl.pallas_call(
    _stage1_body,
    grid=(N // RB,),
    in_specs=[
        pl.BlockSpec((RB, D), lambda i: (i, 0)),
        pl.BlockSpec((D, D), lambda i: (0, 0)),
        pl.BlockSpec((1, D), lambda i: (0, 0)),
        pl.BlockSpec((D, D), lambda i: (0, 0)),
    ],
    out_specs=[
        pl.BlockSpec((RB, D), lambda i: (i, 0)),
        pl.BlockSpec((RB, D), lambda i: (i, 0)),
    ],
    out_shape=[
        jax.ShapeDtypeStruct((N, D), jnp.float32),
        jax.ShapeDtypeStruct((N, D), jnp.float32),
    ],
)


def _stage2_body(acc_ref, b_ref, w2_ref, h_ref, m2_ref):
  s = acc_ref[0] + acc_ref[1] + b_ref[...]
  h = jnp.maximum(s, 0.0)
  h_ref[...] = h
  m2_ref[...] = jnp.dot(h, w2_ref[...], preferred_element_type=jnp.float32)


_stage2 = pl.pallas_call(
    _stage2_body,
    grid=(N // RB,),
    in_specs=[
        pl.BlockSpec((NC, RB, D), lambda i: (0, i, 0)),
        pl.BlockSpec((1, D), lambda i: (0, 0)),
        pl.BlockSpec((D, D), lambda i: (0, 0)),
    ],
    out_specs=[
        pl.BlockSpec((RB, D), lambda i: (i, 0)),
        pl.BlockSpec((RB, D), lambda i: (i, 0)),
    ],
    out_shape=[
        jax.ShapeDtypeStruct((N, D), jnp.float32),
        jax.ShapeDtypeStruct((N, D), jnp.float32),
    ],
)


def _stage3_body(acc_ref, b_ref, h_ref):
  h_ref[...] = jnp.maximum(acc_ref[0] + acc_ref[1] + b_ref[...], 0.0)


_stage3 = pl.pallas_call(
    _stage3_body,
    grid=(N // RB,),
    in_specs=[
        pl.BlockSpec((NC, RB, D), lambda i: (0, i, 0)),
        pl.BlockSpec((1, D), lambda i: (0, 0)),
    ],
    out_specs=pl.BlockSpec((RB, D), lambda i: (i, 0)),
    out_shape=jax.ShapeDtypeStruct((N, D), jnp.float32),
)


def kernel(x, edge_indexs, edge_attrs, W_lin, b_lin, W_convs, b_convs):
  edge_indexs = edge_indexs.astype(jnp.int32)
  edge_attrs = edge_attrs.astype(jnp.float32)
  cols_flat = edge_indexs[:, 1, :].reshape(-1)
  ws_flat = edge_attrs.reshape(-1)

  wbits = jax.lax.bitcast_convert_type(edge_attrs, jnp.int32)
  packed = [
      jnp.stack([edge_indexs[a, 0].reshape(-1, C),
                 edge_indexs[a, 1].reshape(-1, C),
                 wbits[a].reshape(-1, C)], axis=1).reshape(-1)
      for a in range(ALPHA)
  ]

  degp = _deg_kernel(cols_flat, ws_flat)
  dinv = _dinv_tc(degp.reshape(NW, ALPHA * N))

  h0, m1 = _stage1(x, W_lin, b_lin.reshape(1, D), W_convs[0])

  accp1 = _msg_kernels[0](m1, packed[0], dinv).reshape(NC, N, D)

  h1, m2 = _stage2(accp1, b_convs[0].reshape(1, D), W_convs[1])

  accp2 = _msg_kernels[1](m2, packed[1], dinv).reshape(NC, N, D)

  h2 = _stage3(accp2, b_convs[1].reshape(1, D))

  return (h0, h1, h2)


# speedup vs baseline: 32.3427x; 1.1840x over previous
import dataclasses
import functools

import jax
import jax.numpy as jnp
from jax import lax
from jax.experimental import pallas as pl
from jax.experimental.pallas import tpu as pltpu
from jax.experimental.pallas import tpu_sc as plsc

N = 10000
E = 320000
ALPHA = 2
D = 128

NC = 2
NS = 16
NW = NC * NS
L = 16
EPT = E // NW
C = 80
NCHUNK = EPT // C
STRIPE = 632
LAST_STRIPE = N - (NS - 1) * STRIPE

_mesh = plsc.VectorSubcoreMesh(core_axis_name="c", subcore_axis_name="s")

_sc_params = pltpu.CompilerParams()
if "needs_layout_passes" in pltpu.CompilerParams.__dataclass_fields__:
  _sc_params = dataclasses.replace(_sc_params, needs_layout_passes=False)


@functools.partial(
    pl.kernel,
    out_type=jax.ShapeDtypeStruct((NW * ALPHA * N,), jnp.float32),
    mesh=_mesh,
    scratch_types=[
        pltpu.VMEM((ALPHA * N,), jnp.float32),
        pltpu.VMEM((EPT,), jnp.int32),
        pltpu.VMEM((EPT,), jnp.float32),
        pltpu.VMEM((EPT,), jnp.int32),
        pltpu.VMEM((EPT,), jnp.float32),
        pltpu.SemaphoreType.DMA,
        pltpu.SemaphoreType.DMA,
    ],
    compiler_params=_sc_params,
)
def _deg_kernel(eflat_hbm, wflat_hbm, degp_hbm, degb, c0, w0, c1, w1, s0, s1):
  cid = lax.axis_index("c")
  sid = lax.axis_index("s")
  wid = cid * NS + sid

  cbufs, wbufs, sems = (c0, c1), (w0, w1), (s0, s1)
  for a in range(ALPHA):
    pltpu.async_copy(
        eflat_hbm.at[pl.ds((2 * a + 1) * E + wid * EPT, EPT)], cbufs[a],
        sems[a])
    pltpu.async_copy(
        wflat_hbm.at[pl.ds(a * E + wid * EPT, EPT)], wbufs[a], sems[a])

  zero = jnp.zeros((L,), jnp.float32)

  @pl.loop(0, ALPHA * N // L)
  def _(g):
    degb[pl.ds(g * L, L)] = zero

  for a in range(ALPHA):
    pltpu.make_async_copy(eflat_hbm.at[pl.ds(0, EPT)], cbufs[a],
                          sems[a]).wait()
    pltpu.make_async_copy(wflat_hbm.at[pl.ds(0, EPT)], wbufs[a],
                          sems[a]).wait()

    @pl.loop(0, EPT // L)
    def _(g):
      c16 = cbufs[a][pl.ds(g * L, L)] + a * N
      w16 = wbufs[a][pl.ds(g * L, L)]
      plsc.addupdate_scatter(degb, [c16], w16)

  pltpu.sync_copy(degb, degp_hbm.at[pl.ds(wid * ALPHA * N, ALPHA * N)])


def _dinv_body(degp_ref, dinv_ref):
  d = jnp.sum(degp_ref[...], axis=0)
  dinv_ref[...] = jnp.where(d > 0.0, lax.rsqrt(jnp.where(d > 0.0, d, 1.0)), 0.0)


_dinv_tc = pl.pallas_call(
    _dinv_body,
    out_shape=jax.ShapeDtypeStruct((ALPHA * N,), jnp.float32),
)


def _make_msg_kernel(a):
  @functools.partial(
      pl.kernel,
      out_type=jax.ShapeDtypeStruct((NC * N, D), jnp.float32),
      mesh=_mesh,
      scratch_types=[
          pltpu.VMEM((N,), jnp.float32),
          pltpu.VMEM((2 * C,), jnp.int32),
          pltpu.VMEM((2 * C,), jnp.int32),
          pltpu.VMEM((2 * C,), jnp.int32),
          pltpu.VMEM((C,), jnp.float32),
          pltpu.VMEM((C,), jnp.float32),
          pltpu.VMEM((C,), jnp.float32),
          pltpu.VMEM((C, D), jnp.float32),
          pltpu.VMEM((C, D), jnp.float32),
          pltpu.VMEM((C, D), jnp.float32),
          pltpu.VMEM((C,), jnp.int32),
          pltpu.VMEM((C,), jnp.int32),
          pltpu.VMEM((C,), jnp.int32),
          pltpu.VMEM((C,), jnp.int32),
          pltpu.VMEM((C,), jnp.int32),
          pltpu.VMEM((C,), jnp.int32),
          pltpu.VMEM((C,), jnp.float32),
          pltpu.VMEM_SHARED((N, D), jnp.float32),
          pltpu.SemaphoreType.DMA,
          pltpu.SemaphoreType.DMA,
          pltpu.SemaphoreType.DMA,
          pltpu.SemaphoreType.DMA,
          pltpu.SemaphoreType.DMA,
          pltpu.SemaphoreType.DMA,
          pltpu.SemaphoreType.DMA,
          pltpu.SemaphoreType.DMA,
          pltpu.SemaphoreType.DMA,
          pltpu.SemaphoreType.DMA,
      ],
      compiler_params=_sc_params,
  )
  def _msg_kernel(mt_hbm, eflat_hbm, wflat_hbm, dinv_hbm, out_hbm,
                  dinv_v, e0, e1, e2, w0, w1, w2, r0, r1, r2, c0, c1, c2,
                  rw0, rw1, rw2, norm_c, acc_sh,
                  se0, se1, se2, sg0, sg1, sg2, ss0, ss1, ss2, sd):
    cid = lax.axis_index("c")
    sid = lax.axis_index("s")
    wid = cid * NS + sid
    nrows = jnp.where(sid == NS - 1, LAST_STRIPE, STRIPE)
    ebase = wid * EPT

    ebufs, ses = (e0, e1, e2), (se0, se1, se2)
    wbufs = (w0, w1, w2)
    rbufs, sgs = (r0, r1, r2), (sg0, sg1, sg2)
    cbufs, sss = (c0, c1, c2), (ss0, ss1, ss2)
    rwbufs = (rw0, rw1, rw2)

    pltpu.async_copy(dinv_hbm.at[pl.ds(a * N, N)], dinv_v, sd)

    def issue_e(i, k):
      eoff = ebase + i * C
      pltpu.async_copy(
          eflat_hbm.at[pl.ds(2 * a * E + eoff, C)],
          ebufs[k % 3].at[pl.ds(0, C)], ses[k % 3])
      pltpu.async_copy(
          eflat_hbm.at[pl.ds((2 * a + 1) * E + eoff, C)],
          ebufs[k % 3].at[pl.ds(C, C)], ses[k % 3])
      pltpu.async_copy(
          wflat_hbm.at[pl.ds(a * E + eoff, C)], wbufs[k % 3], ses[k % 3])

    def wait_e(k):
      pltpu.make_async_copy(eflat_hbm.at[pl.ds(0, 2 * C)],
                            ebufs[k % 3], ses[k % 3]).wait()
      pltpu.make_async_copy(wflat_hbm.at[pl.ds(0, C)],
                            wbufs[k % 3], ses[k % 3]).wait()

    def stage_rows(k):
      eb, rwb = ebufs[k % 3], rwbufs[k % 3]
      for g in range(C // L):
        rwb[pl.ds(g * L, L)] = eb[pl.ds(g * L, L)]

    def issue_g(k):
      pltpu.async_copy(mt_hbm.at[rwbufs[k % 3]], rbufs[k % 3], sgs[k % 3])

    def wait_g(k):
      pltpu.make_async_copy(mt_hbm.at[rwbufs[k % 3]],
                            rbufs[k % 3], sgs[k % 3]).wait()

    def issue_s(k):
      pltpu.async_copy(rbufs[k % 3], acc_sh.at[cbufs[k % 3]],
                       sss[k % 3], add=True)

    def wait_s(k):
      pltpu.make_async_copy(rbufs[k % 3], acc_sh.at[cbufs[k % 3]],
                            sss[k % 3]).wait()

    def proc_norm(k):
      eb, wb, cb = ebufs[k % 3], wbufs[k % 3], cbufs[k % 3]
      for g in range(C // L):
        r16 = eb[pl.ds(g * L, L)]
        c16 = eb[pl.ds(C + g * L, L)]
        w16 = wb[pl.ds(g * L, L)]
        cb[pl.ds(g * L, L)] = c16
        norm_c[pl.ds(g * L, L)] = (plsc.load_gather(dinv_v, [r16]) * w16
                                   * plsc.load_gather(dinv_v, [c16]))

    def proc_scale(k):
      rb = rbufs[k % 3]

      @pl.loop(0, C)
      def _(e):
        s = plsc.load_gather(norm_c, [jnp.full((L,), e, jnp.int32)])
        for j in range(D // L):
          rb[e, pl.ds(j * L, L)] = rb[e, pl.ds(j * L, L)] * s

    def body(i, k, drain_s=True, next_e=True, next_g=True):
      wait_g(k)
      proc_norm(k)
      if next_e:
        issue_e(i + 3, k)
      proc_scale(k)
      issue_s(k)
      if drain_s:
        wait_s(k + 2)
      if next_g:
        wait_e(k + 2)
        stage_rows(k + 2)
        issue_g(k + 2)

    zero = jnp.zeros((L,), jnp.float32)

    @pl.loop(0, C * D // L)
    def _(g):
      r0[g * L // D, pl.ds((g * L) % D, L)] = zero

    @pl.loop(0, nrows // 8)
    def _(k):
      pltpu.sync_copy(r0.at[pl.ds(0, 8)],
                      acc_sh.at[pl.ds(sid * STRIPE + k * 8, 8)])

    pltpu.make_async_copy(dinv_hbm.at[pl.ds(0, N)], dinv_v, sd).wait()
    plsc.subcore_barrier()

    issue_e(0, 0)
    issue_e(1, 1)
    issue_e(2, 2)
    wait_e(0)
    stage_rows(0)
    issue_g(0)
    wait_e(1)
    stage_rows(1)
    issue_g(1)
    body(0, 0, drain_s=False)
    body(1, 1)

    @pl.loop(0, 40)
    def _(t):
      i0 = 2 + t * 3
      for k in range(3):
        body(i0 + k, 2 + k)

    for i in range(122, NCHUNK):
      body(i, i, next_e=(i + 3 < NCHUNK), next_g=(i + 2 < NCHUNK))
    wait_s(NCHUNK - 1)

    plsc.subcore_barrier()

    pltpu.sync_copy(acc_sh.at[pl.ds(sid * STRIPE, nrows)],
                    out_hbm.at[pl.ds(cid * N + sid * STRIPE, nrows)])

  return _msg_kernel


_msg_kernels = [_make_msg_kernel(a) for a in range(ALPHA)]


RB = 1000


def _stage1_body(x_ref, wl_ref, bl_ref, w1_ref, h0_ref, m1_ref):
  h = jnp.dot(x_ref[...], wl_ref[...], preferred_element_type=jnp.float32)
  h = jnp.maximum(h + bl_ref[...], 0.0)
  h0_ref[...] = h
  m1_ref[...] = jnp.dot(h, w1_ref[...], preferred_element_type=jnp.float32)


_stage1 = pl.pallas_call(
    _stage1_body,
    grid=(N // RB,),
    in_specs=[
        pl.BlockSpec((RB, D), lambda i: (i, 0)),
        pl.BlockSpec((D, D), lambda i: (0, 0)),
        pl.BlockSpec((1, D), lambda i: (0, 0)),
        pl.BlockSpec((D, D), lambda i: (0, 0)),
    ],
    out_specs=[
        pl.BlockSpec((RB, D), lambda i: (i, 0)),
        pl.BlockSpec((RB, D), lambda i: (i, 0)),
    ],
    out_shape=[
        jax.ShapeDtypeStruct((N, D), jnp.float32),
        jax.ShapeDtypeStruct((N, D), jnp.float32),
    ],
)


def _stage2_body(acc_ref, b_ref, w2_ref, h_ref, m2_ref):
  s = acc_ref[0] + acc_ref[1] + b_ref[...]
  h = jnp.maximum(s, 0.0)
  h_ref[...] = h
  m2_ref[...] = jnp.dot(h, w2_ref[...], preferred_element_type=jnp.float32)


_stage2 = pl.pallas_call(
    _stage2_body,
    grid=(N // RB,),
    in_specs=[
        pl.BlockSpec((NC, RB, D), lambda i: (0, i, 0)),
        pl.BlockSpec((1, D), lambda i: (0, 0)),
        pl.BlockSpec((D, D), lambda i: (0, 0)),
    ],
    out_specs=[
        pl.BlockSpec((RB, D), lambda i: (i, 0)),
        pl.BlockSpec((RB, D), lambda i: (i, 0)),
    ],
    out_shape=[
        jax.ShapeDtypeStruct((N, D), jnp.float32),
        jax.ShapeDtypeStruct((N, D), jnp.float32),
    ],
)


def _stage3_body(acc_ref, b_ref, h_ref):
  h_ref[...] = jnp.maximum(acc_ref[0] + acc_ref[1] + b_ref[...], 0.0)


_stage3 = pl.pallas_call(
    _stage3_body,
    grid=(N // RB,),
    in_specs=[
        pl.BlockSpec((NC, RB, D), lambda i: (0, i, 0)),
        pl.BlockSpec((1, D), lambda i: (0, 0)),
    ],
    out_specs=pl.BlockSpec((RB, D), lambda i: (i, 0)),
    out_shape=jax.ShapeDtypeStruct((N, D), jnp.float32),
)


def kernel(x, edge_indexs, edge_attrs, W_lin, b_lin, W_convs, b_convs):
  eflat = edge_indexs.astype(jnp.int32).reshape(-1)
  wflat = edge_attrs.astype(jnp.float32).reshape(-1)

  degp = _deg_kernel(eflat, wflat)
  dinv = _dinv_tc(degp.reshape(NW, ALPHA * N))

  h0, m1 = _stage1(x, W_lin, b_lin.reshape(1, D), W_convs[0])

  accp1 = _msg_kernels[0](m1, eflat, wflat, dinv).reshape(NC, N, D)

  h1, m2 = _stage2(accp1, b_convs[0].reshape(1, D), W_convs[1])

  accp2 = _msg_kernels[1](m2, eflat, wflat, dinv).reshape(NC, N, D)

  h2 = _stage3(accp2, b_convs[1].reshape(1, D))

  return (h0, h1, h2)


# speedup vs baseline: 32.9180x; 1.0178x over previous
import dataclasses
import functools

import jax
import jax.numpy as jnp
from jax import lax
from jax.experimental import pallas as pl
from jax.experimental.pallas import tpu as pltpu
from jax.experimental.pallas import tpu_sc as plsc

N = 10000
E = 320000
ALPHA = 2
D = 128

NC = 2
NS = 16
NW = NC * NS
L = 16
EPT = E // NW
C = 80
NCHUNK = EPT // C
STRIPE = 632
LAST_STRIPE = N - (NS - 1) * STRIPE

_mesh = plsc.VectorSubcoreMesh(core_axis_name="c", subcore_axis_name="s")

_sc_params = pltpu.CompilerParams()
if "needs_layout_passes" in pltpu.CompilerParams.__dataclass_fields__:
  _sc_params = dataclasses.replace(_sc_params, needs_layout_passes=False)


@functools.partial(
    pl.kernel,
    out_type=jax.ShapeDtypeStruct((NW * ALPHA * N,), jnp.float32),
    mesh=_mesh,
    scratch_types=[
        pltpu.VMEM((ALPHA * N,), jnp.float32),
        pltpu.VMEM((EPT,), jnp.int32),
        pltpu.VMEM((EPT,), jnp.float32),
        pltpu.VMEM((EPT,), jnp.int32),
        pltpu.VMEM((EPT,), jnp.float32),
        pltpu.SemaphoreType.DMA,
        pltpu.SemaphoreType.DMA,
    ],
    compiler_params=_sc_params,
)
def _deg_kernel(eflat_hbm, wflat_hbm, degp_hbm, degb, c0, w0, c1, w1, s0, s1):
  cid = lax.axis_index("c")
  sid = lax.axis_index("s")
  wid = cid * NS + sid

  cbufs, wbufs, sems = (c0, c1), (w0, w1), (s0, s1)
  for a in range(ALPHA):
    pltpu.async_copy(
        eflat_hbm.at[pl.ds((2 * a + 1) * E + wid * EPT, EPT)], cbufs[a],
        sems[a])
    pltpu.async_copy(
        wflat_hbm.at[pl.ds(a * E + wid * EPT, EPT)], wbufs[a], sems[a])

  zero = jnp.zeros((L,), jnp.float32)

  @pl.loop(0, ALPHA * N // L)
  def _(g):
    degb[pl.ds(g * L, L)] = zero

  for a in range(ALPHA):
    pltpu.make_async_copy(eflat_hbm.at[pl.ds(0, EPT)], cbufs[a],
                          sems[a]).wait()
    pltpu.make_async_copy(wflat_hbm.at[pl.ds(0, EPT)], wbufs[a],
                          sems[a]).wait()

    @pl.loop(0, EPT // L)
    def _(g):
      c16 = cbufs[a][pl.ds(g * L, L)] + a * N
      w16 = wbufs[a][pl.ds(g * L, L)]
      plsc.addupdate_scatter(degb, [c16], w16)

  pltpu.sync_copy(degb, degp_hbm.at[pl.ds(wid * ALPHA * N, ALPHA * N)])


def _dinv_body(degp_ref, dinv_ref):
  d = jnp.sum(degp_ref[...], axis=0)
  dinv_ref[...] = jnp.where(d > 0.0, lax.rsqrt(jnp.where(d > 0.0, d, 1.0)), 0.0)


_dinv_tc = pl.pallas_call(
    _dinv_body,
    out_shape=jax.ShapeDtypeStruct((ALPHA * N,), jnp.float32),
)


def _make_msg_kernel(a):
  @functools.partial(
      pl.kernel,
      out_type=jax.ShapeDtypeStruct((NC * N, D), jnp.float32),
      mesh=_mesh,
      scratch_types=[
          pltpu.VMEM((N,), jnp.float32),
          pltpu.VMEM((2 * C,), jnp.int32),
          pltpu.VMEM((2 * C,), jnp.int32),
          pltpu.VMEM((2 * C,), jnp.int32),
          pltpu.VMEM((C,), jnp.float32),
          pltpu.VMEM((C,), jnp.float32),
          pltpu.VMEM((C,), jnp.float32),
          pltpu.VMEM((C, D), jnp.float32),
          pltpu.VMEM((C, D), jnp.float32),
          pltpu.VMEM((C, D), jnp.float32),
          pltpu.VMEM((C,), jnp.int32),
          pltpu.VMEM((C,), jnp.int32),
          pltpu.VMEM((C,), jnp.int32),
          pltpu.VMEM((C,), jnp.int32),
          pltpu.VMEM((C,), jnp.int32),
          pltpu.VMEM((C,), jnp.int32),
          pltpu.VMEM((C,), jnp.float32),
          pltpu.VMEM_SHARED((N, D), jnp.float32),
          pltpu.SemaphoreType.DMA,
          pltpu.SemaphoreType.DMA,
          pltpu.SemaphoreType.DMA,
          pltpu.SemaphoreType.DMA,
          pltpu.SemaphoreType.DMA,
          pltpu.SemaphoreType.DMA,
          pltpu.SemaphoreType.DMA,
          pltpu.SemaphoreType.DMA,
          pltpu.SemaphoreType.DMA,
          pltpu.SemaphoreType.DMA,
      ],
      compiler_params=_sc_params,
  )
  def _msg_kernel(mt_hbm, eflat_hbm, wflat_hbm, dinv_hbm, out_hbm,
                  dinv_v, e0, e1, e2, w0, w1, w2, r0, r1, r2, c0, c1, c2,
                  rw0, rw1, rw2, norm_c, acc_sh,
                  se0, se1, se2, sg0, sg1, sg2, ss0, ss1, ss2, sd):
    cid = lax.axis_index("c")
    sid = lax.axis_index("s")
    wid = cid * NS + sid
    nrows = jnp.where(sid == NS - 1, LAST_STRIPE, STRIPE)
    ebase = wid * EPT

    ebufs, ses = (e0, e1, e2), (se0, se1, se2)
    wbufs = (w0, w1, w2)
    rbufs, sgs = (r0, r1, r2), (sg0, sg1, sg2)
    cbufs, sss = (c0, c1, c2), (ss0, ss1, ss2)
    rwbufs = (rw0, rw1, rw2)

    pltpu.async_copy(dinv_hbm.at[pl.ds(a * N, N)], dinv_v, sd)

    def issue_e(i, k):
      eoff = ebase + i * C
      pltpu.async_copy(
          eflat_hbm.at[pl.ds(2 * a * E + eoff, C)],
          ebufs[k % 3].at[pl.ds(0, C)], ses[k % 3])
      pltpu.async_copy(
          eflat_hbm.at[pl.ds((2 * a + 1) * E + eoff, C)],
          ebufs[k % 3].at[pl.ds(C, C)], ses[k % 3])
      pltpu.async_copy(
          wflat_hbm.at[pl.ds(a * E + eoff, C)], wbufs[k % 3], ses[k % 3])

    def wait_e(k):
      pltpu.make_async_copy(eflat_hbm.at[pl.ds(0, 2 * C)],
                            ebufs[k % 3], ses[k % 3]).wait()
      pltpu.make_async_copy(wflat_hbm.at[pl.ds(0, C)],
                            wbufs[k % 3], ses[k % 3]).wait()

    def stage_rows(k):
      eb, rwb = ebufs[k % 3], rwbufs[k % 3]
      for g in range(C // L):
        rwb[pl.ds(g * L, L)] = eb[pl.ds(g * L, L)]

    def issue_g(k):
      pltpu.async_copy(mt_hbm.at[rwbufs[k % 3]], rbufs[k % 3], sgs[k % 3])

    def wait_g(k):
      pltpu.make_async_copy(mt_hbm.at[rwbufs[k % 3]],
                            rbufs[k % 3], sgs[k % 3]).wait()

    def issue_s(k):
      pltpu.async_copy(rbufs[k % 3], acc_sh.at[cbufs[k % 3]],
                       sss[k % 3], add=True)

    def wait_s(k):
      pltpu.make_async_copy(rbufs[k % 3], acc_sh.at[cbufs[k % 3]],
                            sss[k % 3]).wait()

    def proc_norm(k):
      eb, wb, cb = ebufs[k % 3], wbufs[k % 3], cbufs[k % 3]
      for g in range(C // L):
        r16 = eb[pl.ds(g * L, L)]
        c16 = eb[pl.ds(C + g * L, L)]
        w16 = wb[pl.ds(g * L, L)]
        cb[pl.ds(g * L, L)] = c16
        norm_c[pl.ds(g * L, L)] = (plsc.load_gather(dinv_v, [r16]) * w16
                                   * plsc.load_gather(dinv_v, [c16]))

    def proc_scale(k):
      rb = rbufs[k % 3]

      @pl.loop(0, C, unroll=4)
      def _(e):
        s = plsc.load_gather(norm_c, [jnp.full((L,), e, jnp.int32)])
        for j in range(D // L):
          rb[e, pl.ds(j * L, L)] = rb[e, pl.ds(j * L, L)] * s

    def body(i, k, drain_s=True, next_e=True, next_g=True):
      wait_g(k)
      proc_norm(k)
      if next_e:
        issue_e(i + 3, k)
      proc_scale(k)
      issue_s(k)
      if drain_s:
        wait_s(k + 2)
      if next_g:
        wait_e(k + 2)
        stage_rows(k + 2)
        issue_g(k + 2)

    zero = jnp.zeros((L,), jnp.float32)

    @pl.loop(0, C * D // L)
    def _(g):
      r0[g * L // D, pl.ds((g * L) % D, L)] = zero

    @pl.loop(0, nrows // 8)
    def _(k):
      pltpu.sync_copy(r0.at[pl.ds(0, 8)],
                      acc_sh.at[pl.ds(sid * STRIPE + k * 8, 8)])

    pltpu.make_async_copy(dinv_hbm.at[pl.ds(0, N)], dinv_v, sd).wait()
    plsc.subcore_barrier()

    issue_e(0, 0)
    issue_e(1, 1)
    issue_e(2, 2)
    wait_e(0)
    stage_rows(0)
    issue_g(0)
    wait_e(1)
    stage_rows(1)
    issue_g(1)
    body(0, 0, drain_s=False)
    body(1, 1)

    @pl.loop(0, 40)
    def _(t):
      i0 = 2 + t * 3
      for k in range(3):
        body(i0 + k, 2 + k)

    for i in range(122, NCHUNK):
      body(i, i, next_e=(i + 3 < NCHUNK), next_g=(i + 2 < NCHUNK))
    wait_s(NCHUNK - 1)

    plsc.subcore_barrier()

    pltpu.sync_copy(acc_sh.at[pl.ds(sid * STRIPE, nrows)],
                    out_hbm.at[pl.ds(cid * N + sid * STRIPE, nrows)])

  return _msg_kernel


_msg_kernels = [_make_msg_kernel(a) for a in range(ALPHA)]


RB = 1000


def _stage1_body(x_ref, wl_ref, bl_ref, w1_ref, h0_ref, m1_ref):
  h = jnp.dot(x_ref[...], wl_ref[...], preferred_element_type=jnp.float32)
  h = jnp.maximum(h + bl_ref[...], 0.0)
  h0_ref[...] = h
  m1_ref[...] = jnp.dot(h, w1_ref[...], preferred_element_type=jnp.float32)


_stage1 = pl.pallas_call(
    _stage1_body,
    grid=(N // RB,),
    in_specs=[
        pl.BlockSpec((RB, D), lambda i: (i, 0)),
        pl.BlockSpec((D, D), lambda i: (0, 0)),
        pl.BlockSpec((1, D), lambda i: (0, 0)),
        pl.BlockSpec((D, D), lambda i: (0, 0)),
    ],
    out_specs=[
        pl.BlockSpec((RB, D), lambda i: (i, 0)),
        pl.BlockSpec((RB, D), lambda i: (i, 0)),
    ],
    out_shape=[
        jax.ShapeDtypeStruct((N, D), jnp.float32),
        jax.ShapeDtypeStruct((N, D), jnp.float32),
    ],
)


def _stage2_body(acc_ref, b_ref, w2_ref, h_ref, m2_ref):
  s = acc_ref[0] + acc_ref[1] + b_ref[...]
  h = jnp.maximum(s, 0.0)
  h_ref[...] = h
  m2_ref[...] = jnp.dot(h, w2_ref[...], preferred_element_type=jnp.float32)


_stage2 = pl.pallas_call(
    _stage2_body,
    grid=(N // RB,),
    in_specs=[
        pl.BlockSpec((NC, RB, D), lambda i: (0, i, 0)),
        pl.BlockSpec((1, D), lambda i: (0, 0)),
        pl.BlockSpec((D, D), lambda i: (0, 0)),
    ],
    out_specs=[
        pl.BlockSpec((RB, D), lambda i: (i, 0)),
        pl.BlockSpec((RB, D), lambda i: (i, 0)),
    ],
    out_shape=[
        jax.ShapeDtypeStruct((N, D), jnp.float32),
        jax.ShapeDtypeStruct((N, D), jnp.float32),
    ],
)


def _stage3_body(acc_ref, b_ref, h_ref):
  h_ref[...] = jnp.maximum(acc_ref[0] + acc_ref[1] + b_ref[...], 0.0)


_stage3 = pl.pallas_call(
    _stage3_body,
    grid=(N // RB,),
    in_specs=[
        pl.BlockSpec((NC, RB, D), lambda i: (0, i, 0)),
        pl.BlockSpec((1, D), lambda i: (0, 0)),
    ],
    out_specs=pl.BlockSpec((RB, D), lambda i: (i, 0)),
    out_shape=jax.ShapeDtypeStruct((N, D), jnp.float32),
)


def kernel(x, edge_indexs, edge_attrs, W_lin, b_lin, W_convs, b_convs):
  eflat = edge_indexs.astype(jnp.int32).reshape(-1)
  wflat = edge_attrs.astype(jnp.float32).reshape(-1)

  degp = _deg_kernel(eflat, wflat)
  dinv = _dinv_tc(degp.reshape(NW, ALPHA * N))

  h0, m1 = _stage1(x, W_lin, b_lin.reshape(1, D), W_convs[0])

  accp1 = _msg_kernels[0](m1, eflat, wflat, dinv).reshape(NC, N, D)

  h1, m2 = _stage2(accp1, b_convs[0].reshape(1, D), W_convs[1])

  accp2 = _msg_kernels[1](m2, eflat, wflat, dinv).reshape(NC, N, D)

  h2 = _stage3(accp2, b_convs[1].reshape(1, D))

  return (h0, h1, h2)


# speedup vs baseline: 33.7208x; 1.0244x over previous
import dataclasses
import functools

import jax
import jax.numpy as jnp
from jax import lax
from jax.experimental import pallas as pl
from jax.experimental.pallas import tpu as pltpu
from jax.experimental.pallas import tpu_sc as plsc

N = 10000
E = 320000
ALPHA = 2
D = 128

NC = 2
NS = 16
NW = NC * NS
L = 16
EPT = E // NW
C = 80
NCHUNK = EPT // C
STRIPE = 632
LAST_STRIPE = N - (NS - 1) * STRIPE

_mesh = plsc.VectorSubcoreMesh(core_axis_name="c", subcore_axis_name="s")

_sc_params = pltpu.CompilerParams()
if "needs_layout_passes" in pltpu.CompilerParams.__dataclass_fields__:
  _sc_params = dataclasses.replace(_sc_params, needs_layout_passes=False)


@functools.partial(
    pl.kernel,
    out_type=jax.ShapeDtypeStruct((NW * ALPHA * N,), jnp.float32),
    mesh=_mesh,
    scratch_types=[
        pltpu.VMEM((ALPHA * N,), jnp.float32),
        pltpu.VMEM((EPT,), jnp.int32),
        pltpu.VMEM((EPT,), jnp.float32),
        pltpu.VMEM((EPT,), jnp.int32),
        pltpu.VMEM((EPT,), jnp.float32),
        pltpu.SemaphoreType.DMA,
        pltpu.SemaphoreType.DMA,
    ],
    compiler_params=_sc_params,
)
def _deg_kernel(eflat_hbm, wflat_hbm, degp_hbm, degb, c0, w0, c1, w1, s0, s1):
  cid = lax.axis_index("c")
  sid = lax.axis_index("s")
  wid = cid * NS + sid

  cbufs, wbufs, sems = (c0, c1), (w0, w1), (s0, s1)
  for a in range(ALPHA):
    pltpu.async_copy(
        eflat_hbm.at[pl.ds((2 * a + 1) * E + wid * EPT, EPT)], cbufs[a],
        sems[a])
    pltpu.async_copy(
        wflat_hbm.at[pl.ds(a * E + wid * EPT, EPT)], wbufs[a], sems[a])

  zero = jnp.zeros((L,), jnp.float32)

  @pl.loop(0, ALPHA * N // L, unroll=8)
  def _(g):
    degb[pl.ds(g * L, L)] = zero

  for a in range(ALPHA):
    pltpu.make_async_copy(eflat_hbm.at[pl.ds(0, EPT)], cbufs[a],
                          sems[a]).wait()
    pltpu.make_async_copy(wflat_hbm.at[pl.ds(0, EPT)], wbufs[a],
                          sems[a]).wait()

    @pl.loop(0, EPT // L, unroll=8)
    def _(g):
      c16 = cbufs[a][pl.ds(g * L, L)] + a * N
      w16 = wbufs[a][pl.ds(g * L, L)]
      plsc.addupdate_scatter(degb, [c16], w16)

  pltpu.sync_copy(degb, degp_hbm.at[pl.ds(wid * ALPHA * N, ALPHA * N)])


def _dinv_body(degp_ref, dinv_ref):
  d = jnp.sum(degp_ref[...], axis=0)
  dinv_ref[...] = jnp.where(d > 0.0, lax.rsqrt(jnp.where(d > 0.0, d, 1.0)), 0.0)


_dinv_tc = pl.pallas_call(
    _dinv_body,
    out_shape=jax.ShapeDtypeStruct((ALPHA * N,), jnp.float32),
)


def _make_msg_kernel(a):
  @functools.partial(
      pl.kernel,
      out_type=jax.ShapeDtypeStruct((NC * N, D), jnp.float32),
      mesh=_mesh,
      scratch_types=[
          pltpu.VMEM((N,), jnp.float32),
          pltpu.VMEM((2 * C,), jnp.int32),
          pltpu.VMEM((2 * C,), jnp.int32),
          pltpu.VMEM((2 * C,), jnp.int32),
          pltpu.VMEM((C,), jnp.float32),
          pltpu.VMEM((C,), jnp.float32),
          pltpu.VMEM((C,), jnp.float32),
          pltpu.VMEM((C, D), jnp.float32),
          pltpu.VMEM((C, D), jnp.float32),
          pltpu.VMEM((C, D), jnp.float32),
          pltpu.VMEM((C,), jnp.int32),
          pltpu.VMEM((C,), jnp.int32),
          pltpu.VMEM((C,), jnp.int32),
          pltpu.VMEM((C,), jnp.int32),
          pltpu.VMEM((C,), jnp.int32),
          pltpu.VMEM((C,), jnp.int32),
          pltpu.VMEM((C,), jnp.float32),
          pltpu.VMEM_SHARED((N, D), jnp.float32),
          pltpu.SemaphoreType.DMA,
          pltpu.SemaphoreType.DMA,
          pltpu.SemaphoreType.DMA,
          pltpu.SemaphoreType.DMA,
          pltpu.SemaphoreType.DMA,
          pltpu.SemaphoreType.DMA,
          pltpu.SemaphoreType.DMA,
          pltpu.SemaphoreType.DMA,
          pltpu.SemaphoreType.DMA,
          pltpu.SemaphoreType.DMA,
      ],
      compiler_params=_sc_params,
  )
  def _msg_kernel(mt_hbm, eflat_hbm, wflat_hbm, dinv_hbm, out_hbm,
                  dinv_v, e0, e1, e2, w0, w1, w2, r0, r1, r2, c0, c1, c2,
                  rw0, rw1, rw2, norm_c, acc_sh,
                  se0, se1, se2, sg0, sg1, sg2, ss0, ss1, ss2, sd):
    cid = lax.axis_index("c")
    sid = lax.axis_index("s")
    wid = cid * NS + sid
    nrows = jnp.where(sid == NS - 1, LAST_STRIPE, STRIPE)
    ebase = wid * EPT

    ebufs, ses = (e0, e1, e2), (se0, se1, se2)
    wbufs = (w0, w1, w2)
    rbufs, sgs = (r0, r1, r2), (sg0, sg1, sg2)
    cbufs, sss = (c0, c1, c2), (ss0, ss1, ss2)
    rwbufs = (rw0, rw1, rw2)

    pltpu.async_copy(dinv_hbm.at[pl.ds(a * N, N)], dinv_v, sd)

    def issue_e(i, k):
      eoff = ebase + i * C
      pltpu.async_copy(
          eflat_hbm.at[pl.ds(2 * a * E + eoff, C)],
          ebufs[k % 3].at[pl.ds(0, C)], ses[k % 3])
      pltpu.async_copy(
          eflat_hbm.at[pl.ds((2 * a + 1) * E + eoff, C)],
          ebufs[k % 3].at[pl.ds(C, C)], ses[k % 3])
      pltpu.async_copy(
          wflat_hbm.at[pl.ds(a * E + eoff, C)], wbufs[k % 3], ses[k % 3])

    def wait_e(k):
      pltpu.make_async_copy(eflat_hbm.at[pl.ds(0, 2 * C)],
                            ebufs[k % 3], ses[k % 3]).wait()
      pltpu.make_async_copy(wflat_hbm.at[pl.ds(0, C)],
                            wbufs[k % 3], ses[k % 3]).wait()

    def stage_rows(k):
      eb, rwb = ebufs[k % 3], rwbufs[k % 3]
      for g in range(C // L):
        rwb[pl.ds(g * L, L)] = eb[pl.ds(g * L, L)]

    def issue_g(k):
      pltpu.async_copy(mt_hbm.at[rwbufs[k % 3]], rbufs[k % 3], sgs[k % 3])

    def wait_g(k):
      pltpu.make_async_copy(mt_hbm.at[rwbufs[k % 3]],
                            rbufs[k % 3], sgs[k % 3]).wait()

    def issue_s(k):
      pltpu.async_copy(rbufs[k % 3], acc_sh.at[cbufs[k % 3]],
                       sss[k % 3], add=True)

    def wait_s(k):
      pltpu.make_async_copy(rbufs[k % 3], acc_sh.at[cbufs[k % 3]],
                            sss[k % 3]).wait()

    def proc_norm(k):
      eb, wb, cb = ebufs[k % 3], wbufs[k % 3], cbufs[k % 3]
      for g in range(C // L):
        r16 = eb[pl.ds(g * L, L)]
        c16 = eb[pl.ds(C + g * L, L)]
        w16 = wb[pl.ds(g * L, L)]
        cb[pl.ds(g * L, L)] = c16
        norm_c[pl.ds(g * L, L)] = (plsc.load_gather(dinv_v, [r16]) * w16
                                   * plsc.load_gather(dinv_v, [c16]))

    def proc_scale(k):
      rb = rbufs[k % 3]

      @pl.loop(0, C, unroll=4)
      def _(e):
        s = plsc.load_gather(norm_c, [jnp.full((L,), e, jnp.int32)])
        for j in range(D // L):
          rb[e, pl.ds(j * L, L)] = rb[e, pl.ds(j * L, L)] * s

    def body(i, k, drain_s=True, next_e=True, next_g=True):
      wait_g(k)
      proc_norm(k)
      if next_e:
        issue_e(i + 3, k)
      proc_scale(k)
      issue_s(k)
      if drain_s:
        wait_s(k + 2)
      if next_g:
        wait_e(k + 2)
        stage_rows(k + 2)
        issue_g(k + 2)

    zero = jnp.zeros((L,), jnp.float32)

    @pl.loop(0, C * D // L)
    def _(g):
      r0[g * L // D, pl.ds((g * L) % D, L)] = zero

    @pl.loop(0, nrows // 8)
    def _(k):
      pltpu.sync_copy(r0.at[pl.ds(0, 8)],
                      acc_sh.at[pl.ds(sid * STRIPE + k * 8, 8)])

    pltpu.make_async_copy(dinv_hbm.at[pl.ds(0, N)], dinv_v, sd).wait()
    plsc.subcore_barrier()

    issue_e(0, 0)
    issue_e(1, 1)
    issue_e(2, 2)
    wait_e(0)
    stage_rows(0)
    issue_g(0)
    wait_e(1)
    stage_rows(1)
    issue_g(1)
    body(0, 0, drain_s=False)
    body(1, 1)

    @pl.loop(0, 40)
    def _(t):
      i0 = 2 + t * 3
      for k in range(3):
        body(i0 + k, 2 + k)

    for i in range(122, NCHUNK):
      body(i, i, next_e=(i + 3 < NCHUNK), next_g=(i + 2 < NCHUNK))
    wait_s(NCHUNK - 1)

    plsc.subcore_barrier()

    pltpu.sync_copy(acc_sh.at[pl.ds(sid * STRIPE, nrows)],
                    out_hbm.at[pl.ds(cid * N + sid * STRIPE, nrows)])

  return _msg_kernel


_msg_kernels = [_make_msg_kernel(a) for a in range(ALPHA)]


RB = 2000


def _stage1_body(x_ref, wl_ref, bl_ref, w1_ref, h0_ref, m1_ref):
  h = jnp.dot(x_ref[...], wl_ref[...], preferred_element_type=jnp.float32)
  h = jnp.maximum(h + bl_ref[...], 0.0)
  h0_ref[...] = h
  m1_ref[...] = jnp.dot(h, w1_ref[...], preferred_element_type=jnp.float32)


_stage1 = pl.pallas_call(
    _stage1_body,
    grid=(N // RB,),
    in_specs=[
        pl.BlockSpec((RB, D), lambda i: (i, 0)),
        pl.BlockSpec((D, D), lambda i: (0, 0)),
        pl.BlockSpec((1, D), lambda i: (0, 0)),
        pl.BlockSpec((D, D), lambda i: (0, 0)),
    ],
    out_specs=[
        pl.BlockSpec((RB, D), lambda i: (i, 0)),
        pl.BlockSpec((RB, D), lambda i: (i, 0)),
    ],
    out_shape=[
        jax.ShapeDtypeStruct((N, D), jnp.float32),
        jax.ShapeDtypeStruct((N, D), jnp.float32),
    ],
)


def _stage2_body(acc_ref, b_ref, w2_ref, h_ref, m2_ref):
  s = acc_ref[0] + acc_ref[1] + b_ref[...]
  h = jnp.maximum(s, 0.0)
  h_ref[...] = h
  m2_ref[...] = jnp.dot(h, w2_ref[...], preferred_element_type=jnp.float32)


_stage2 = pl.pallas_call(
    _stage2_body,
    grid=(N // RB,),
    in_specs=[
        pl.BlockSpec((NC, RB, D), lambda i: (0, i, 0)),
        pl.BlockSpec((1, D), lambda i: (0, 0)),
        pl.BlockSpec((D, D), lambda i: (0, 0)),
    ],
    out_specs=[
        pl.BlockSpec((RB, D), lambda i: (i, 0)),
        pl.BlockSpec((RB, D), lambda i: (i, 0)),
    ],
    out_shape=[
        jax.ShapeDtypeStruct((N, D), jnp.float32),
        jax.ShapeDtypeStruct((N, D), jnp.float32),
    ],
)


def _stage3_body(acc_ref, b_ref, h_ref):
  h_ref[...] = jnp.maximum(acc_ref[0] + acc_ref[1] + b_ref[...], 0.0)


_stage3 = pl.pallas_call(
    _stage3_body,
    grid=(N // RB,),
    in_specs=[
        pl.BlockSpec((NC, RB, D), lambda i: (0, i, 0)),
        pl.BlockSpec((1, D), lambda i: (0, 0)),
    ],
    out_specs=pl.BlockSpec((RB, D), lambda i: (i, 0)),
    out_shape=jax.ShapeDtypeStruct((N, D), jnp.float32),
)


def kernel(x, edge_indexs, edge_attrs, W_lin, b_lin, W_convs, b_convs):
  eflat = edge_indexs.astype(jnp.int32).reshape(-1)
  wflat = edge_attrs.astype(jnp.float32).reshape(-1)

  degp = _deg_kernel(eflat, wflat)
  dinv = _dinv_tc(degp.reshape(NW, ALPHA * N))

  h0, m1 = _stage1(x, W_lin, b_lin.reshape(1, D), W_convs[0])

  accp1 = _msg_kernels[0](m1, eflat, wflat, dinv).reshape(NC, N, D)

  h1, m2 = _stage2(accp1, b_convs[0].reshape(1, D), W_convs[1])

  accp2 = _msg_kernels[1](m2, eflat, wflat, dinv).reshape(NC, N, D)

  h2 = _stage3(accp2, b_convs[1].reshape(1, D))

  return (h0, h1, h2)
